# Initial kernel scaffold; baseline (speedup 1.0000x reference)
import functools

import jax
import jax.numpy as jnp
from jax import lax
from jax.experimental import pallas as pl
from jax.experimental.pallas import tpu as pltpu
from jax.experimental.pallas import tpu_sc as plsc

N_NODES = 10000
D = 128
E = 320000
NP = 10240
DUMMY = N_NODES
NC = 2
NS = 16
NW = NC * NS
K = 128
NCHUNK = 79
EPW = K * NCHUNK
ROWS_PER_TILE = NP // NS
DEGW = 16

_mesh = plsc.VectorSubcoreMesh(core_axis_name="c", subcore_axis_name="s")


def _masked_rows(rowv, colv, idxv):
    for j in range(K // 16):
        sl = pl.ds(j * 16, 16)
        r = rowv[sl]
        c = colv[sl]
        idxv[sl] = jnp.where(r != c, r, DUMMY)


@functools.partial(
    pl.kernel,
    mesh=_mesh,
    out_type=jax.ShapeDtypeStruct((NC, NP, DEGW), jnp.float32),
    scratch_types=[
        pltpu.VMEM((K,), jnp.int32),
        pltpu.VMEM((K,), jnp.int32),
        pltpu.VMEM((K,), jnp.int32),
        pltpu.VMEM_SHARED((NP, DEGW), jnp.float32),
    ],
)
def _deg_kernel(row_hbm, col_hbm, ones_hbm, zeros_hbm, deg_hbm,
                rowv, colv, idxv, acc_sh):
    c = lax.axis_index("c")
    s = lax.axis_index("s")
    wid = s * NC + c
    pltpu.sync_copy(zeros_hbm, acc_sh.at[pl.ds(s * ROWS_PER_TILE, ROWS_PER_TILE)])
    plsc.subcore_barrier()

    def chunk(i, carry):
        pltpu.sync_copy(row_hbm.at[wid, i], rowv)
        pltpu.sync_copy(col_hbm.at[wid, i], colv)
        _masked_rows(rowv, colv, idxv)
        pltpu.sync_copy(ones_hbm, acc_sh.at[idxv], add=True)
        return carry

    lax.fori_loop(0, NCHUNK, chunk, 0)
    plsc.subcore_barrier()
    sl = pl.ds(s * ROWS_PER_TILE, ROWS_PER_TILE)
    pltpu.sync_copy(acc_sh.at[sl], deg_hbm.at[c, sl])


@functools.partial(
    pl.kernel,
    mesh=_mesh,
    out_type=jax.ShapeDtypeStruct((NC, NP, D), jnp.float32),
    scratch_types=[
        pltpu.VMEM((K,), jnp.int32),
        pltpu.VMEM((K,), jnp.int32),
        pltpu.VMEM((K,), jnp.int32),
        pltpu.VMEM((K, D), jnp.float32),
        pltpu.VMEM_SHARED((NP, D), jnp.float32),
        pltpu.SemaphoreType.DMA,
    ],
)
def _prop_kernel(h_hbm, row_hbm, col_hbm, zeros_hbm, p_hbm,
                 rowv, colv, idxv, rows_v, acc_sh, sem):
    c = lax.axis_index("c")
    s = lax.axis_index("s")
    wid = s * NC + c
    for t in range(ROWS_PER_TILE // K):
        pltpu.sync_copy(zeros_hbm,
                        acc_sh.at[pl.ds(s * ROWS_PER_TILE + t * K, K)])
    plsc.subcore_barrier()

    def chunk(i, carry):
        pltpu.sync_copy(row_hbm.at[wid, i], rowv)
        pltpu.sync_copy(col_hbm.at[wid, i], colv)
        _masked_rows(rowv, colv, idxv)
        pltpu.async_copy(h_hbm.at[colv], rows_v, sem).wait()
        pltpu.sync_copy(rows_v, acc_sh.at[idxv], add=True)
        return carry

    lax.fori_loop(0, NCHUNK, chunk, 0)
    plsc.subcore_barrier()
    for t in range(ROWS_PER_TILE // K):
        sl = pl.ds(s * ROWS_PER_TILE + t * K, K)
        pltpu.sync_copy(acc_sh.at[sl], p_hbm.at[c, sl])


BLK = 1280


def _tc1_body(x_ref, w_ref, d0_ref, d1_ref, o_ref):
    deg = d0_ref[:, :1] + d1_ref[:, :1] + 1.0
    z = jnp.dot(x_ref[...], w_ref[...], preferred_element_type=jnp.float32)
    o_ref[...] = z * lax.rsqrt(deg)


def _make_combine(use_rsqrt):
    def body(p0_ref, p1_ref, c_ref, d0_ref, d1_ref, o_ref):
        deg = d0_ref[:, :1] + d1_ref[:, :1] + 1.0
        scale = lax.rsqrt(deg) if use_rsqrt else 1.0 / deg
        o_ref[...] = (p0_ref[...] + p1_ref[...] + c_ref[...]) * scale
    return body


_row_spec = pl.BlockSpec((BLK, D), lambda i: (i, 0))
_deg_spec = pl.BlockSpec((BLK, DEGW), lambda i: (i, 0))


def _tc1(xp, W, d0, d1):
    return pl.pallas_call(
        _tc1_body,
        grid=(NP // BLK,),
        in_specs=[_row_spec, pl.BlockSpec((D, D), lambda i: (0, 0)),
                  _deg_spec, _deg_spec],
        out_specs=_row_spec,
        out_shape=jax.ShapeDtypeStruct((NP, D), jnp.float32),
    )(xp, W, d0, d1)


def _combine(p0, p1, cc, d0, d1, use_rsqrt):
    return pl.pallas_call(
        _make_combine(use_rsqrt),
        grid=(NP // BLK,),
        in_specs=[_row_spec, _row_spec, _row_spec, _deg_spec, _deg_spec],
        out_specs=_row_spec,
        out_shape=jax.ShapeDtypeStruct((NP, D), jnp.float32),
    )(p0, p1, cc, d0, d1)


def kernel(edge_index, x, W):
    row = edge_index[0].astype(jnp.int32)
    col = edge_index[1].astype(jnp.int32)
    pad = NW * EPW - E
    rowp = jnp.pad(row, (0, pad)).reshape(NW, NCHUNK, K)
    colp = jnp.pad(col, (0, pad)).reshape(NW, NCHUNK, K)
    xp = jnp.pad(x, ((0, NP - N_NODES), (0, 0)))
    ones_deg = jnp.ones((K, DEGW), jnp.float32)
    zeros_deg = jnp.zeros((ROWS_PER_TILE, DEGW), jnp.float32)
    zeros_rows = jnp.zeros((K, D), jnp.float32)

    degp = _deg_kernel(rowp, colp, ones_deg, zeros_deg)
    d0 = degp[0]
    d1 = degp[1]
    t1 = _tc1(xp, W, d0, d1)
    p = _prop_kernel(t1, rowp, colp, zeros_rows)
    t2 = _combine(p[0], p[1], t1, d0, d1, use_rsqrt=False)
    p2 = _prop_kernel(t2, rowp, colp, zeros_rows)
    out = _combine(p2[0], p2[1], t2, d0, d1, use_rsqrt=True)
    return out[:N_NODES]


# speedup vs baseline: 10.3298x; 10.3298x over previous
import functools

import jax
import jax.numpy as jnp
from jax import lax
from jax.experimental import pallas as pl
from jax.experimental.pallas import tpu as pltpu
from jax.experimental.pallas import tpu_sc as plsc

N_NODES = 10000
D = 128
E = 320000
NP = 10240
DUMMY = N_NODES
NC = 2
NS = 16
NW = NC * NS
K = 128
NCHUNK = 79
EPW = K * NCHUNK
ROWS_PER_TILE = NP // NS
DEGW = 128

_mesh = plsc.VectorSubcoreMesh(core_axis_name="c", subcore_axis_name="s")


def _masked_rows(rowv, colv, idxv):
    for j in range(K // 16):
        sl = pl.ds(j * 16, 16)
        r = rowv[sl]
        c = colv[sl]
        idxv[sl] = jnp.where(r != c, r, DUMMY)


@functools.partial(
    pl.kernel,
    mesh=_mesh,
    out_type=jax.ShapeDtypeStruct((NC, NP, DEGW), jnp.float32),
    scratch_types=[
        pltpu.VMEM((K,), jnp.int32),
        pltpu.VMEM((K,), jnp.int32),
        pltpu.VMEM((K,), jnp.int32),
        pltpu.VMEM((K, DEGW), jnp.float32),
        pltpu.VMEM_SHARED((NP, DEGW), jnp.float32),
    ],
)
def _deg_kernel(row_hbm, col_hbm, ones_hbm, zeros_hbm, deg_hbm,
                rowv, colv, idxv, ones_v, acc_sh):
    c = lax.axis_index("c")
    s = lax.axis_index("s")
    wid = s * NC + c
    pltpu.sync_copy(ones_hbm, ones_v)
    pltpu.sync_copy(zeros_hbm, acc_sh.at[pl.ds(s * ROWS_PER_TILE, ROWS_PER_TILE)])
    plsc.subcore_barrier()

    def chunk(i, carry):
        pltpu.sync_copy(row_hbm.at[wid, i], rowv)
        pltpu.sync_copy(col_hbm.at[wid, i], colv)
        _masked_rows(rowv, colv, idxv)
        pltpu.sync_copy(ones_v, acc_sh.at[idxv], add=True)
        return carry

    lax.fori_loop(0, NCHUNK, chunk, 0)
    plsc.subcore_barrier()
    sl = pl.ds(s * ROWS_PER_TILE, ROWS_PER_TILE)
    pltpu.sync_copy(acc_sh.at[sl], deg_hbm.at[c, sl])


@functools.partial(
    pl.kernel,
    mesh=_mesh,
    out_type=jax.ShapeDtypeStruct((NC, NP, D), jnp.float32),
    scratch_types=[
        pltpu.VMEM((K,), jnp.int32),
        pltpu.VMEM((K,), jnp.int32),
        pltpu.VMEM((K,), jnp.int32),
        pltpu.VMEM((K, D), jnp.float32),
        pltpu.VMEM_SHARED((NP, D), jnp.float32),
        pltpu.SemaphoreType.DMA,
    ],
)
def _prop_kernel(h_hbm, row_hbm, col_hbm, zeros_hbm, p_hbm,
                 rowv, colv, idxv, rows_v, acc_sh, sem):
    c = lax.axis_index("c")
    s = lax.axis_index("s")
    wid = s * NC + c
    for t in range(ROWS_PER_TILE // K):
        pltpu.sync_copy(zeros_hbm,
                        acc_sh.at[pl.ds(s * ROWS_PER_TILE + t * K, K)])
    plsc.subcore_barrier()

    def chunk(i, carry):
        pltpu.sync_copy(row_hbm.at[wid, i], rowv)
        pltpu.sync_copy(col_hbm.at[wid, i], colv)
        _masked_rows(rowv, colv, idxv)
        pltpu.async_copy(h_hbm.at[colv], rows_v, sem).wait()
        pltpu.sync_copy(rows_v, acc_sh.at[idxv], add=True)
        return carry

    lax.fori_loop(0, NCHUNK, chunk, 0)
    plsc.subcore_barrier()
    for t in range(ROWS_PER_TILE // K):
        sl = pl.ds(s * ROWS_PER_TILE + t * K, K)
        pltpu.sync_copy(acc_sh.at[sl], p_hbm.at[c, sl])


BLK = 1280


def _tc1_body(x_ref, w_ref, d0_ref, d1_ref, o_ref):
    deg = d0_ref[:, :1] + d1_ref[:, :1] + 1.0
    z = jnp.dot(x_ref[...], w_ref[...], preferred_element_type=jnp.float32)
    o_ref[...] = z * lax.rsqrt(deg)


def _make_combine(use_rsqrt):
    def body(p0_ref, p1_ref, c_ref, d0_ref, d1_ref, o_ref):
        deg = d0_ref[:, :1] + d1_ref[:, :1] + 1.0
        scale = lax.rsqrt(deg) if use_rsqrt else 1.0 / deg
        o_ref[...] = (p0_ref[...] + p1_ref[...] + c_ref[...]) * scale
    return body


_row_spec = pl.BlockSpec((BLK, D), lambda i: (i, 0))
_deg_spec = pl.BlockSpec((BLK, DEGW), lambda i: (i, 0))


def _tc1(xp, W, d0, d1):
    return pl.pallas_call(
        _tc1_body,
        grid=(NP // BLK,),
        in_specs=[_row_spec, pl.BlockSpec((D, D), lambda i: (0, 0)),
                  _deg_spec, _deg_spec],
        out_specs=_row_spec,
        out_shape=jax.ShapeDtypeStruct((NP, D), jnp.float32),
    )(xp, W, d0, d1)


def _combine(p0, p1, cc, d0, d1, use_rsqrt):
    return pl.pallas_call(
        _make_combine(use_rsqrt),
        grid=(NP // BLK,),
        in_specs=[_row_spec, _row_spec, _row_spec, _deg_spec, _deg_spec],
        out_specs=_row_spec,
        out_shape=jax.ShapeDtypeStruct((NP, D), jnp.float32),
    )(p0, p1, cc, d0, d1)


def kernel(edge_index, x, W):
    row = edge_index[0].astype(jnp.int32)
    col = edge_index[1].astype(jnp.int32)
    pad = NW * EPW - E
    rowp = jnp.pad(row, (0, pad)).reshape(NW, NCHUNK, K)
    colp = jnp.pad(col, (0, pad)).reshape(NW, NCHUNK, K)
    xp = jnp.pad(x, ((0, NP - N_NODES), (0, 0)))
    ones_deg = jnp.ones((K, DEGW), jnp.float32)
    zeros_deg = jnp.zeros((ROWS_PER_TILE, DEGW), jnp.float32)
    zeros_rows = jnp.zeros((K, D), jnp.float32)

    degp = _deg_kernel(rowp, colp, ones_deg, zeros_deg)
    d0 = degp[0]
    d1 = degp[1]
    t1 = _tc1(xp, W, d0, d1)
    p = _prop_kernel(t1, rowp, colp, zeros_rows)
    t2 = _combine(p[0], p[1], t1, d0, d1, use_rsqrt=False)
    p2 = _prop_kernel(t2, rowp, colp, zeros_rows)
    out = _combine(p2[0], p2[1], t2, d0, d1, use_rsqrt=True)
    return out[:N_NODES]
